# baseline (speedup 1.0000x reference)
import jax
import jax.numpy as jnp
from jax import lax
from jax.experimental import pallas as pl
from jax.experimental.pallas import tpu as pltpu
from jax.experimental.pallas import tpu_sc as plsc

_INFO = plsc.get_sparse_core_info()
_NC, _NS = _INFO.num_cores, _INFO.num_subcores
_NW = _NC * _NS

_ROWS = 4096
_SEQ = 50
_RPW = _ROWS // _NW
_EMB = 128

_NB = 8
_NPHASE = _RPW // _NB
_BIGROWS = _NB * _SEQ


def _embed_body(table_hbm, idx_hbm, out_hbm, idx_v, big0, big1, gsem, ssem):
    big = (big0, big1)
    wid = lax.axis_index("s") * _NC + lax.axis_index("c")
    base = wid * _RPW
    pltpu.sync_copy(idx_hbm.at[pl.ds(base, _RPW)], idx_v)
    obase = wid * (_RPW * _SEQ)

    for b in range(_NB):
        pltpu.async_copy(table_hbm.at[idx_v.at[b]],
                         big[0].at[pl.ds(b * _SEQ, _SEQ)], gsem.at[0, b])

    for p in range(_NPHASE):
        h = p % 2
        nxt = 1 - h
        for b in range(_NB):
            pltpu.make_async_copy(
                table_hbm.at[idx_v.at[p * _NB + b]],
                big[h].at[pl.ds(b * _SEQ, _SEQ)], gsem.at[h, b]
            ).wait()
        if p + 1 < _NPHASE:
            if p >= 1:
                oprev = pl.multiple_of(obase + (p - 1) * _BIGROWS, 8)
                pltpu.make_async_copy(
                    big[nxt], out_hbm.at[pl.ds(oprev, _BIGROWS)], ssem.at[nxt]
                ).wait()
            for b in range(_NB):
                pltpu.async_copy(table_hbm.at[idx_v.at[(p + 1) * _NB + b]],
                                 big[nxt].at[pl.ds(b * _SEQ, _SEQ)],
                                 gsem.at[nxt, b])
        ocur = pl.multiple_of(obase + p * _BIGROWS, 8)
        pltpu.async_copy(big[h], out_hbm.at[pl.ds(ocur, _BIGROWS)],
                         ssem.at[h])

    for p in (_NPHASE - 2, _NPHASE - 1):
        h = p % 2
        op_ = pl.multiple_of(obase + p * _BIGROWS, 8)
        pltpu.make_async_copy(
            big[h], out_hbm.at[pl.ds(op_, _BIGROWS)], ssem.at[h]
        ).wait()


@jax.jit
def _embed(input_seq, table):
    mesh = plsc.VectorSubcoreMesh(core_axis_name="c", subcore_axis_name="s")
    fn = pl.kernel(
        _embed_body,
        mesh=mesh,
        out_type=jax.ShapeDtypeStruct((_ROWS * _SEQ, _EMB), jnp.float32),
        scratch_types=[
            pltpu.VMEM((_RPW, _SEQ), jnp.int32),
            pltpu.VMEM((_BIGROWS, _EMB), jnp.float32),
            pltpu.VMEM((_BIGROWS, _EMB), jnp.float32),
            pltpu.SemaphoreType.DMA((2, _NB)),
            pltpu.SemaphoreType.DMA((2,)),
        ],
    )
    out = fn(table, input_seq)
    return out.reshape(_ROWS, _SEQ, _EMB)


def kernel(input_seq, table):
    return _embed(input_seq, table)


# speedup vs baseline: 1.0036x; 1.0036x over previous
import jax
import jax.numpy as jnp
from jax import lax
from jax.experimental import pallas as pl
from jax.experimental.pallas import tpu as pltpu
from jax.experimental.pallas import tpu_sc as plsc

_INFO = plsc.get_sparse_core_info()
_NC, _NS = _INFO.num_cores, _INFO.num_subcores
_NW = _NC * _NS

_ROWS = 4096
_SEQ = 50
_CHUNK = 100
_NCHUNKS = _ROWS * _SEQ // _CHUNK
_CPW = _NCHUNKS // _NW
_EMB = 128

_NB = 8
_NGROUPS = _CPW // _NB


def _embed_body(table_hbm, idx_hbm, out_hbm, idx_v, *bufs_and_sems):
    rows = bufs_and_sems[:_NB]
    gsem, ssem = bufs_and_sems[_NB], bufs_and_sems[_NB + 1]
    wid = lax.axis_index("s") * _NC + lax.axis_index("c")
    base = wid * _CPW
    pltpu.sync_copy(idx_hbm.at[pl.ds(base, _CPW)], idx_v)

    for b in range(_NB):
        pltpu.async_copy(table_hbm.at[idx_v.at[b]], rows[b], gsem.at[b])

    def group(g, carry):
        j0 = g * _NB
        for b in range(_NB):
            pltpu.make_async_copy(
                table_hbm.at[idx_v.at[j0 + b]], rows[b], gsem.at[b]
            ).wait()
            pltpu.async_copy(rows[b], out_hbm.at[base + j0 + b], ssem.at[b])
        for b in range(_NB):
            pltpu.make_async_copy(
                rows[b], out_hbm.at[base + j0 + b], ssem.at[b]
            ).wait()
            jn = jnp.minimum(j0 + _NB + b, _CPW - 1)

            @pl.when(g + 1 < _NGROUPS)
            def _():
                pltpu.async_copy(table_hbm.at[idx_v.at[jn]], rows[b],
                                 gsem.at[b])

        return carry

    lax.fori_loop(0, _NGROUPS, group, 0)


@jax.jit
def _embed(input_seq, table):
    mesh = plsc.VectorSubcoreMesh(core_axis_name="c", subcore_axis_name="s")
    fn = pl.kernel(
        _embed_body,
        mesh=mesh,
        out_type=jax.ShapeDtypeStruct((_NCHUNKS, _CHUNK, _EMB), jnp.float32),
        scratch_types=[
            pltpu.VMEM((_CPW, _CHUNK), jnp.int32),
        ]
        + [pltpu.VMEM((_CHUNK, _EMB), jnp.float32) for _ in range(_NB)]
        + [
            pltpu.SemaphoreType.DMA((_NB,)),
            pltpu.SemaphoreType.DMA((_NB,)),
        ],
    )
    idx = input_seq.reshape(_NCHUNKS, _CHUNK)
    out = fn(table, idx)
    return out.reshape(_ROWS, _SEQ, _EMB)


def kernel(input_seq, table):
    return _embed(input_seq, table)


# speedup vs baseline: 1.7896x; 1.7831x over previous
import jax
import jax.numpy as jnp
from jax import lax
from jax.experimental import pallas as pl
from jax.experimental.pallas import tpu as pltpu
from jax.experimental.pallas import tpu_sc as plsc

_INFO = plsc.get_sparse_core_info()
_NC, _NS = _INFO.num_cores, _INFO.num_subcores
_NW = _NC * _NS

_ROWS = 4096
_SEQ = 50
_RPW = _ROWS // _NW
_EMB = 128

_NB = 8
_NPHASE = _RPW // _NB


def _embed_body(table_hbm, idx_hbm, out_hbm, idx_v, big0, big1, gsem, ssem):
    big = (big0, big1)
    wid = lax.axis_index("s") * _NC + lax.axis_index("c")
    base = wid * _RPW
    pltpu.sync_copy(idx_hbm.at[pl.ds(base, _RPW)], idx_v)

    def scatter_dst(p):
        off = pl.multiple_of(base + p * _NB, 8)
        return out_hbm.at[pl.ds(off, _NB)]

    for b in range(_NB):
        pltpu.async_copy(table_hbm.at[idx_v.at[b]], big[0].at[b],
                         gsem.at[0, b])

    for p in range(_NPHASE):
        h = p % 2
        nxt = 1 - h
        for b in range(_NB):
            pltpu.make_async_copy(
                table_hbm.at[idx_v.at[p * _NB + b]], big[h].at[b],
                gsem.at[h, b]
            ).wait()
        if p + 1 < _NPHASE:
            if p >= 1:
                pltpu.make_async_copy(
                    big[nxt], scatter_dst(p - 1), ssem.at[nxt]
                ).wait()
            for b in range(_NB):
                pltpu.async_copy(table_hbm.at[idx_v.at[(p + 1) * _NB + b]],
                                 big[nxt].at[b], gsem.at[nxt, b])
        pltpu.async_copy(big[h], scatter_dst(p), ssem.at[h])

    for p in (_NPHASE - 2, _NPHASE - 1):
        pltpu.make_async_copy(big[p % 2], scatter_dst(p),
                              ssem.at[p % 2]).wait()


@jax.jit
def _embed(input_seq, table):
    mesh = plsc.VectorSubcoreMesh(core_axis_name="c", subcore_axis_name="s")
    fn = pl.kernel(
        _embed_body,
        mesh=mesh,
        out_type=jax.ShapeDtypeStruct((_ROWS, _SEQ, _EMB), jnp.float32),
        scratch_types=[
            pltpu.VMEM((_RPW, _SEQ), jnp.int32),
            pltpu.VMEM((_NB, _SEQ, _EMB), jnp.float32),
            pltpu.VMEM((_NB, _SEQ, _EMB), jnp.float32),
            pltpu.SemaphoreType.DMA((2, _NB)),
            pltpu.SemaphoreType.DMA((2,)),
        ],
    )
    return fn(table, input_seq)


def kernel(input_seq, table):
    return _embed(input_seq, table)


# speedup vs baseline: 1.8005x; 1.0061x over previous
import jax
import jax.numpy as jnp
from jax import lax
from jax.experimental import pallas as pl
from jax.experimental.pallas import tpu as pltpu
from jax.experimental.pallas import tpu_sc as plsc

_INFO = plsc.get_sparse_core_info()
_NC, _NS = _INFO.num_cores, _INFO.num_subcores
_NW = _NC * _NS

_ROWS = 4096
_SEQ = 50
_RPW = _ROWS // _NW
_EMB = 128

_NB = 8
_NGROUPS = _RPW // _NB


def _embed_body(table_hbm, idx_hbm, out_hbm, idx_v, *bufs_and_sems):
    rows = bufs_and_sems[:_NB]
    gsem, ssem = bufs_and_sems[_NB], bufs_and_sems[_NB + 1]
    wid = lax.axis_index("s") * _NC + lax.axis_index("c")
    base = wid * _RPW
    pltpu.sync_copy(idx_hbm.at[pl.ds(base, _RPW)], idx_v)

    for b in range(_NB):
        pltpu.async_copy(table_hbm.at[idx_v.at[b]], rows[b], gsem.at[b])

    def group(g, carry):
        j0 = g * _NB
        for b in range(_NB):
            pltpu.make_async_copy(
                table_hbm.at[idx_v.at[j0 + b]], rows[b], gsem.at[b]
            ).wait()
            pltpu.async_copy(rows[b], out_hbm.at[base + j0 + b], ssem.at[b])
        for b in range(_NB):
            pltpu.make_async_copy(
                rows[b], out_hbm.at[base + j0 + b], ssem.at[b]
            ).wait()
            jn = jnp.minimum(j0 + _NB + b, _RPW - 1)

            @pl.when(g + 1 < _NGROUPS)
            def _():
                pltpu.async_copy(table_hbm.at[idx_v.at[jn]], rows[b],
                                 gsem.at[b])

        return carry

    lax.fori_loop(0, _NGROUPS, group, 0)


@jax.jit
def _embed(input_seq, table):
    mesh = plsc.VectorSubcoreMesh(core_axis_name="c", subcore_axis_name="s")
    fn = pl.kernel(
        _embed_body,
        mesh=mesh,
        out_type=jax.ShapeDtypeStruct((_ROWS, _SEQ, _EMB), jnp.float32),
        scratch_types=[
            pltpu.VMEM((_RPW, _SEQ), jnp.int32),
        ]
        + [pltpu.VMEM((_SEQ, _EMB), jnp.float32) for _ in range(_NB)]
        + [
            pltpu.SemaphoreType.DMA((_NB,)),
            pltpu.SemaphoreType.DMA((_NB,)),
        ],
    )
    return fn(table, input_seq)


def kernel(input_seq, table):
    return _embed(input_seq, table)
